# baseline (speedup 1.0000x reference)
import jax
import jax.numpy as jnp
from jax.experimental import pallas as pl

N = 10000
E = 160000
F_IN = 50
HID = 256
H1 = 4
H3 = 6
C_OUT = 121


def _matmul_tc(x, w):
    n, k = x.shape
    _, m = w.shape
    bn = 512

    def body(x_ref, w_ref, o_ref):
        o_ref[...] = jnp.dot(x_ref[...], w_ref[...],
                             preferred_element_type=jnp.float32)

    grid = (pl.cdiv(n, bn),)
    return pl.pallas_call(
        body,
        grid=grid,
        in_specs=[
            pl.BlockSpec((bn, k), lambda i: (i, 0)),
            pl.BlockSpec((k, m), lambda i: (0, 0)),
        ],
        out_specs=pl.BlockSpec((bn, m), lambda i: (i, 0)),
        out_shape=jax.ShapeDtypeStruct((n, m), jnp.float32),
    )(x, w)


def _gat(x, src, dst, W, a_src, a_dst, b, heads, ch, concat):
    n = x.shape[0]
    h = _matmul_tc(x, W).reshape(n, heads, ch)
    es = jnp.sum(h * a_src, axis=-1)
    ed = jnp.sum(h * a_dst, axis=-1)
    e = jax.nn.leaky_relu(es[src] + ed[dst], negative_slope=0.2)
    emax = jax.ops.segment_max(e, dst, num_segments=n)
    emax = jnp.where(jnp.isfinite(emax), emax, 0.0)
    ex = jnp.exp(e - emax[dst])
    den = jax.ops.segment_sum(ex, dst, num_segments=n)
    alpha = ex / (den[dst] + 1e-16)
    out = jax.ops.segment_sum(h[src] * alpha[:, :, None], dst, num_segments=n)
    if concat:
        return out.reshape(n, heads * ch) + b
    return jnp.mean(out, axis=1) + b


def kernel(x, edge_index, W1, a_src1, a_dst1, b1, Wl1, bl1, W2, a_src2,
           a_dst2, b2, Wl2, bl2, W3, a_src3, a_dst3, b3, Wl3, bl3):
    src = edge_index[0]
    dst = edge_index[1]
    x1 = jax.nn.elu(_gat(x, src, dst, W1, a_src1, a_dst1, b1, H1, HID, True)
                    + _matmul_tc(x, Wl1) + bl1)
    x2 = jax.nn.elu(_gat(x1, src, dst, W2, a_src2, a_dst2, b2, H1, HID, True)
                    + _matmul_tc(x1, Wl2) + bl2)
    x3 = (_gat(x2, src, dst, W3, a_src3, a_dst3, b3, H3, C_OUT, False)
          + _matmul_tc(x2, Wl3) + bl3)
    return x3


# speedup vs baseline: 13.5484x; 13.5484x over previous
import functools

import jax
import jax.numpy as jnp
from jax import lax
from jax.experimental import pallas as pl
from jax.experimental.pallas import tpu as pltpu
from jax.experimental.pallas import tpu_sc as plsc

N = 10000
E = 160000
F_IN = 50
HID = 256
H1 = 4
H3 = 6
C_OUT = 121

L = 16
NW = 32
BD = 64
NBLK = (N + BD - 1) // BD
N_PAD = NBLK * BD
RP_LEN = N_PAD + 80
EP = 163840
EPW = EP // NW
EC = 512
W_WIN = 2048

_mesh = functools.partial(
    plsc.VectorSubcoreMesh, core_axis_name="c", subcore_axis_name="s")


def _iota():
    return lax.iota(jnp.int32, L)


def _lane_i(v, j):
    return jnp.sum(jnp.where(_iota() == j, v, 0))


def _lane_f(v, j):
    return jnp.sum(jnp.where(_iota() == j, v, jnp.float32(0.0)))


def _wid():
    return lax.axis_index("s") * 2 + lax.axis_index("c")


def _edge_scores(es, ed, srcs_pad, dsts_pad, hg):
    nvec = EC // L

    def body(es_hbm, ed_hbm, src_hbm, dst_hbm, out_hbm,
             es_t, ed_t, src_c, dst_c, esc_o):
        w = _wid()
        base = w * EPW
        pltpu.sync_copy(es_hbm, es_t)
        pltpu.sync_copy(ed_hbm, ed_t)

        def chunk(c, _):
            off = pl.multiple_of(base + c * EC, 512)
            pltpu.sync_copy(src_hbm.at[pl.ds(off, EC)], src_c)
            pltpu.sync_copy(dst_hbm.at[pl.ds(off, EC)], dst_c)
            for v in range(nvec):
                s = src_c[pl.ds(v * L, L)]
                d = dst_c[pl.ds(v * L, L)]
                dc = jnp.minimum(d, N - 1)
                for k in range(hg):
                    a = plsc.load_gather(es_t, [s * hg + k])
                    b = plsc.load_gather(ed_t, [dc * hg + k])
                    e = a + b
                    ev = jnp.where(e > 0, e, e * jnp.float32(0.2))
                    esc_o[pl.ds(k * EC + v * L, L)] = ev
            for k in range(hg):
                pltpu.sync_copy(esc_o.at[pl.ds(k * EC, EC)],
                                out_hbm.at[pl.ds(k * EP + off, EC)])
            return _

        lax.fori_loop(0, EPW // EC, chunk, 0)

    f = pl.kernel(
        body,
        out_type=jax.ShapeDtypeStruct((hg * EP,), jnp.float32),
        mesh=_mesh(),
        compiler_params=pltpu.CompilerParams(needs_layout_passes=False),
        scratch_types=[
            pltpu.VMEM((N * hg,), jnp.float32),
            pltpu.VMEM((N * hg,), jnp.float32),
            pltpu.VMEM((EC,), jnp.int32),
            pltpu.VMEM((EC,), jnp.int32),
            pltpu.VMEM((hg * EC,), jnp.float32),
        ],
    )
    return f(es.reshape(-1), ed.reshape(-1), srcs_pad, dsts_pad)


def _message(feat, escore, rp_pad, srcs_pad, dsts_pad, h, din, dout, mode):
    ngrp = BD // L
    neg = jnp.float32(-1e30)

    def body(feat_hbm, esc_hbm, rp_hbm, src_hbm, dst_hbm, out_hbm,
             acc, esc_w, src_w, dst_w, rp_b, mx_t, dn_t, rows, sem):
        w = _wid()

        def block(bi, _):
            blk = w + bi * NW
            d0 = pl.multiple_of(blk * BD, BD)

            def z(i, _):
                acc[pl.ds(i * L, L)] = jnp.zeros((L,), jnp.float32)
                return _
            lax.fori_loop(0, BD * dout // L, z, 0)
            for i in range(BD * h // L):
                mx_t[pl.ds(i * L, L)] = jnp.full((L,), neg)
                dn_t[pl.ds(i * L, L)] = jnp.zeros((L,), jnp.float32)

            pltpu.sync_copy(rp_hbm.at[pl.ds(d0, 80)], rp_b)
            e0 = _lane_i(rp_b[pl.ds(0, L)], 0)
            e1 = _lane_i(rp_b[pl.ds(64, L)], 0)
            e0a = pl.multiple_of(e0 & ~7, 8)
            n_win = (e1 - e0a + W_WIN - 1) // W_WIN

            def stage_esc(wi):
                ws = pl.multiple_of(e0a + wi * W_WIN, 8)
                for k in range(h):
                    pltpu.sync_copy(esc_hbm.at[pl.ds(k * EP + ws, W_WIN)],
                                    esc_w.at[pl.ds(k * W_WIN, W_WIN)])
                return ws

            def win1(wi, _):
                ws = stage_esc(wi)
                for g in range(ngrp):
                    lo = plsc.load_gather(rp_b, [g * L + _iota()])
                    hi = plsc.load_gather(rp_b, [g * L + 1 + _iota()])
                    deg = hi - lo
                    mdeg = jnp.max(deg)

                    def jb(j, mxs):
                        local = lo + j - ws
                        m = (j < deg) & (local >= 0) & (local < W_WIN)
                        lidx = jnp.clip(local, 0, W_WIN - 1)
                        out = []
                        for k in range(h):
                            sv = plsc.load_gather(esc_w, [k * W_WIN + lidx])
                            out.append(
                                jnp.maximum(mxs[k], jnp.where(m, sv, neg)))
                        return tuple(out)

                    mxs = lax.fori_loop(0, mdeg, jb,
                                        tuple(jnp.full((L,), neg)
                                              for _ in range(h)))
                    for k in range(h):
                        sl = pl.ds(k * BD + g * L, L)
                        mx_t[sl] = jnp.maximum(mx_t[sl], mxs[k])
                return _
            lax.fori_loop(0, n_win, win1, 0)

            def win2(wi, _):
                ws = stage_esc(wi)
                for g in range(ngrp):
                    lo = plsc.load_gather(rp_b, [g * L + _iota()])
                    hi = plsc.load_gather(rp_b, [g * L + 1 + _iota()])
                    deg = hi - lo
                    mdeg = jnp.max(deg)
                    mxk = [mx_t[pl.ds(k * BD + g * L, L)] for k in range(h)]

                    def jb(j, dns):
                        local = lo + j - ws
                        m = (j < deg) & (local >= 0) & (local < W_WIN)
                        lidx = jnp.clip(local, 0, W_WIN - 1)
                        out = []
                        for k in range(h):
                            sv = plsc.load_gather(esc_w, [k * W_WIN + lidx])
                            ex = jnp.exp(sv - mxk[k])
                            out.append(dns[k] +
                                       jnp.where(m, ex, jnp.float32(0.0)))
                        return tuple(out)

                    dns = lax.fori_loop(0, mdeg, jb,
                                        tuple(jnp.zeros((L,), jnp.float32)
                                              for _ in range(h)))
                    for k in range(h):
                        sl = pl.ds(k * BD + g * L, L)
                        dn_t[sl] = dn_t[sl] + dns[k]
                return _
            lax.fori_loop(0, n_win, win2, 0)

            def win3(wi, _):
                ws = stage_esc(wi)
                pltpu.sync_copy(src_hbm.at[pl.ds(ws, W_WIN)], src_w)
                pltpu.sync_copy(dst_hbm.at[pl.ds(ws, W_WIN)], dst_w)
                n_ch = jnp.clip((e1 - ws + L - 1) // L, 0, W_WIN // L)

                def chunk(c, _):
                    dv = dst_w[pl.ds(c * L, L)]
                    dloc = dv - d0
                    valid = (dloc >= 0) & (dloc < BD)
                    dc = jnp.clip(dloc, 0, BD - 1)
                    als = []
                    for k in range(h):
                        sv = esc_w[pl.ds(k * W_WIN + c * L, L)]
                        mx = plsc.load_gather(mx_t, [k * BD + dc])
                        dn = plsc.load_gather(dn_t, [k * BD + dc])
                        al = jnp.exp(sv - mx) / (dn + jnp.float32(1e-16))
                        als.append(jnp.where(valid, al, jnp.float32(0.0)))
                    sv_src = src_w[pl.ds(c * L, L)]
                    pltpu.async_copy(feat_hbm.at[sv_src], rows, sem).wait()

                    def rrow(je, off):
                        return plsc.load_gather(
                            rows, [jnp.full((L,), je, jnp.int32),
                                   off + _iota()])

                    def je_body(je, _):
                        dj = _lane_i(dc, je)
                        obase = dj * dout
                        if mode == 2:
                            for k in range(h):
                                a = _lane_f(als[k], je)
                                for f in range(HID // L):
                                    r = rrow(je, k * HID + f * L)
                                    plsc.addupdate(
                                        acc.at[pl.ds(
                                            obase + k * HID + f * L, L)],
                                        a * r)
                        elif mode == 1:
                            rr = [rrow(je, f * L) for f in range(4)]
                            for k in range(h):
                                a = _lane_f(als[k], je)
                                for f in range(4):
                                    plsc.addupdate(
                                        acc.at[pl.ds(
                                            obase + k * 64 + f * L, L)],
                                        a * rr[f])
                        else:
                            avs = [_lane_f(als[k], je) for k in range(h)]
                            tail = jnp.where(_iota() < 9, jnp.float32(1.0),
                                             jnp.float32(0.0))
                            for f in range(8):
                                s = jnp.zeros((L,), jnp.float32)
                                for k in range(h):
                                    r = rrow(je, k * C_OUT + f * L)
                                    s = s + avs[k] * r
                                if f == 7:
                                    s = s * tail
                                plsc.addupdate(
                                    acc.at[pl.ds(obase + f * L, L)], s)
                        return _

                    lax.fori_loop(0, L, je_body, 0)
                    return _

                lax.fori_loop(0, n_ch, chunk, 0)
                return _
            lax.fori_loop(0, n_win, win3, 0)

            pltpu.sync_copy(acc, out_hbm.at[pl.ds(pl.multiple_of(d0 * dout, 128), BD * dout)])
            return _

        nb = NBLK // NW + jnp.where(w < NBLK % NW, 1, 0)
        lax.fori_loop(0, nb, block, 0)

    f = pl.kernel(
        body,
        out_type=jax.ShapeDtypeStruct((N_PAD * dout,), jnp.float32),
        mesh=_mesh(),
        compiler_params=pltpu.CompilerParams(needs_layout_passes=False),
        scratch_types=[
            pltpu.VMEM((BD * dout,), jnp.float32),
            pltpu.VMEM((h * W_WIN,), jnp.float32),
            pltpu.VMEM((W_WIN,), jnp.int32),
            pltpu.VMEM((W_WIN,), jnp.int32),
            pltpu.VMEM((80,), jnp.int32),
            pltpu.VMEM((h * BD,), jnp.float32),
            pltpu.VMEM((h * BD,), jnp.float32),
            pltpu.VMEM((L, din), jnp.float32),
            pltpu.SemaphoreType.DMA,
        ],
    )
    return f(feat, escore, rp_pad, srcs_pad, dsts_pad)


def _k0(x128, uv1p):
    def body(x_ref, w_ref, o_ref):
        o_ref[...] = jnp.dot(x_ref[...], w_ref[...],
                             preferred_element_type=jnp.float32)
    return pl.pallas_call(
        body,
        grid=(10,),
        in_specs=[pl.BlockSpec((1000, 128), lambda i: (i, 0)),
                  pl.BlockSpec((128, 128), lambda i: (0, 0))],
        out_specs=pl.BlockSpec((1000, 128), lambda i: (i, 0)),
        out_shape=jax.ShapeDtypeStruct((N, 128), jnp.float32),
    )(x128, uv1p)


def _ka(g1p, x128, B1p, Wl1p, bias1, W2, uv2p, Wl2):
    def body(g_ref, x_ref, b1_ref, wl1_ref, bias_ref, w2_ref, uv_ref,
             wl2_ref, h2_ref, sc_ref, sk_ref):
        x1 = jnp.dot(g_ref[...], b1_ref[...],
                     preferred_element_type=jnp.float32)
        x1 = x1 + jnp.dot(x_ref[...], wl1_ref[...],
                          preferred_element_type=jnp.float32)
        x1 = x1 + bias_ref[...]
        x1 = jnp.where(x1 > 0, x1, jnp.exp(jnp.minimum(x1, 0.0)) - 1.0)
        h2_ref[...] = jnp.dot(x1, w2_ref[...],
                              preferred_element_type=jnp.float32)
        sc_ref[...] = jnp.dot(x1, uv_ref[...],
                              preferred_element_type=jnp.float32)
        sk_ref[...] = jnp.dot(x1, wl2_ref[...],
                              preferred_element_type=jnp.float32)

    bn = 1000
    return pl.pallas_call(
        body,
        grid=(N // bn,),
        in_specs=[pl.BlockSpec((bn, 256), lambda i: (i, 0)),
                  pl.BlockSpec((bn, 128), lambda i: (i, 0)),
                  pl.BlockSpec((256, 1024), lambda i: (0, 0)),
                  pl.BlockSpec((128, 1024), lambda i: (0, 0)),
                  pl.BlockSpec((1, 1024), lambda i: (0, 0)),
                  pl.BlockSpec((1024, 1024), lambda i: (0, 0)),
                  pl.BlockSpec((1024, 128), lambda i: (0, 0)),
                  pl.BlockSpec((1024, 1024), lambda i: (0, 0))],
        out_specs=[pl.BlockSpec((bn, 1024), lambda i: (i, 0)),
                   pl.BlockSpec((bn, 128), lambda i: (i, 0)),
                   pl.BlockSpec((bn, 1024), lambda i: (i, 0))],
        out_shape=[jax.ShapeDtypeStruct((N, 1024), jnp.float32),
                   jax.ShapeDtypeStruct((N, 128), jnp.float32),
                   jax.ShapeDtypeStruct((N, 1024), jnp.float32)],
    )(g1p, x128, B1p, Wl1p, bias1, W2, uv2p, Wl2)


def _kb(m2, sk2, bias2, W3p, uv3p, Wl3p):
    def body(m_ref, sk_ref, bias_ref, w3_ref, uv_ref, wl3_ref,
             h3_ref, sc_ref, sk3_ref):
        x2 = m_ref[...] + sk_ref[...] + bias_ref[...]
        x2 = jnp.where(x2 > 0, x2, jnp.exp(jnp.minimum(x2, 0.0)) - 1.0)
        h3_ref[...] = jnp.dot(x2, w3_ref[...],
                              preferred_element_type=jnp.float32)
        sc_ref[...] = jnp.dot(x2, uv_ref[...],
                              preferred_element_type=jnp.float32)
        sk3_ref[...] = jnp.dot(x2, wl3_ref[...],
                               preferred_element_type=jnp.float32)

    bn = 1000
    return pl.pallas_call(
        body,
        grid=(N // bn,),
        in_specs=[pl.BlockSpec((bn, 1024), lambda i: (i, 0)),
                  pl.BlockSpec((bn, 1024), lambda i: (i, 0)),
                  pl.BlockSpec((1, 1024), lambda i: (0, 0)),
                  pl.BlockSpec((1024, 768), lambda i: (0, 0)),
                  pl.BlockSpec((1024, 128), lambda i: (0, 0)),
                  pl.BlockSpec((1024, 128), lambda i: (0, 0))],
        out_specs=[pl.BlockSpec((bn, 768), lambda i: (i, 0)),
                   pl.BlockSpec((bn, 128), lambda i: (i, 0)),
                   pl.BlockSpec((bn, 128), lambda i: (i, 0))],
        out_shape=[jax.ShapeDtypeStruct((N, 768), jnp.float32),
                   jax.ShapeDtypeStruct((N, 128), jnp.float32),
                   jax.ShapeDtypeStruct((N, 128), jnp.float32)],
    )(m2, sk2, bias2, W3p, uv3p, Wl3p)


def _kc(m3, sk3, bias3):
    def body(m_ref, sk_ref, b_ref, o_ref):
        v = (m_ref[...] * jnp.float32(1.0 / H3) + sk_ref[...] + b_ref[...])
        o_ref[...] = v[:, :C_OUT]

    bn = 1000
    return pl.pallas_call(
        body,
        grid=(N // bn,),
        in_specs=[pl.BlockSpec((bn, 128), lambda i: (i, 0)),
                  pl.BlockSpec((bn, 128), lambda i: (i, 0)),
                  pl.BlockSpec((1, 128), lambda i: (0, 0))],
        out_specs=pl.BlockSpec((bn, C_OUT), lambda i: (i, 0)),
        out_shape=jax.ShapeDtypeStruct((N, C_OUT), jnp.float32),
    )(m3, sk3, bias3)


def kernel(x, edge_index, W1, a_src1, a_dst1, b1, Wl1, bl1, W2, a_src2,
           a_dst2, b2, Wl2, bl2, W3, a_src3, a_dst3, b3, Wl3, bl3):
    src = edge_index[0]
    dst = edge_index[1]

    perm = jnp.argsort(dst)
    srcs = src[perm]
    dsts = dst[perm]
    srcs_pad = jnp.zeros((EP,), jnp.int32).at[:E].set(srcs)
    dsts_pad = jnp.full((EP,), 2 * N_PAD, jnp.int32).at[:E].set(dsts)
    rp_pad = jnp.searchsorted(dsts, jnp.arange(RP_LEN),
                              side="left").astype(jnp.int32)

    def fold(W, a_s, a_d, heads, ch):
        Wr = W.reshape(W.shape[0], heads, ch)
        u = jnp.einsum("fkc,kc->fk", Wr, a_s)
        v = jnp.einsum("fkc,kc->fk", Wr, a_d)
        return jnp.concatenate([u, v], axis=1)

    uv1 = fold(W1, a_src1, a_dst1, H1, HID)
    uv2 = fold(W2, a_src2, a_dst2, H1, HID)
    uv3 = fold(W3, a_src3, a_dst3, H3, C_OUT)
    uv1p = jnp.zeros((128, 128), jnp.float32).at[:F_IN, :2 * H1].set(uv1)
    uv2p = jnp.zeros((1024, 128), jnp.float32).at[:, :2 * H1].set(uv2)
    uv3p = jnp.zeros((1024, 128), jnp.float32).at[:, :2 * H3].set(uv3)

    x128 = jnp.zeros((N, 128), jnp.float32).at[:, :F_IN].set(x)
    B1p = jnp.zeros((256, 1024), jnp.float32)
    W1r = W1.reshape(F_IN, H1, HID)
    for k in range(H1):
        B1p = B1p.at[k * 64:k * 64 + F_IN,
                     k * HID:(k + 1) * HID].set(W1r[:, k, :])
    Wl1p = jnp.zeros((128, 1024), jnp.float32).at[:F_IN, :].set(Wl1)
    bias1 = (b1 + bl1).reshape(1, 1024)
    bias2 = (b2 + bl2).reshape(1, 1024)
    W3p = jnp.zeros((1024, 768), jnp.float32).at[:, :H3 * C_OUT].set(W3)
    Wl3p = jnp.zeros((1024, 128), jnp.float32).at[:, :C_OUT].set(Wl3)
    bias3 = jnp.zeros((1, 128), jnp.float32).at[0, :C_OUT].set(b3 + bl3)

    sc1 = _k0(x128, uv1p)
    esc1 = _edge_scores(sc1[:, :H1], sc1[:, H1:2 * H1], srcs_pad, dsts_pad,
                        H1)
    g1 = _message(x128, esc1, rp_pad, srcs_pad, dsts_pad, H1, 128, 256, 1)
    g1p = g1[:N * 256].reshape(N, 256)

    h2, sc2, sk2 = _ka(g1p, x128, B1p, Wl1p, bias1, W2, uv2p, Wl2)
    esc2 = _edge_scores(sc2[:, :H1], sc2[:, H1:2 * H1], srcs_pad, dsts_pad,
                        H1)
    m2 = _message(h2, esc2, rp_pad, srcs_pad, dsts_pad, H1, 1024, 1024, 2)
    m2 = m2[:N * 1024].reshape(N, 1024)

    h3, sc3, sk3 = _kb(m2, sk2, bias2, W3p, uv3p, Wl3p)
    esc3a = _edge_scores(sc3[:, :3], sc3[:, H3:H3 + 3], srcs_pad, dsts_pad, 3)
    esc3b = _edge_scores(sc3[:, 3:H3], sc3[:, H3 + 3:2 * H3], srcs_pad,
                         dsts_pad, 3)
    esc3 = jnp.concatenate([esc3a, esc3b], axis=0)
    m3 = _message(h3, esc3, rp_pad, srcs_pad, dsts_pad, H3, 768, 128, 3)
    m3 = m3[:N * 128].reshape(N, 128)

    return _kc(m3, sk3, bias3)


# speedup vs baseline: 16.7603x; 1.2371x over previous
import functools

import jax
import jax.numpy as jnp
from jax import lax
from jax.experimental import pallas as pl
from jax.experimental.pallas import tpu as pltpu
from jax.experimental.pallas import tpu_sc as plsc

N = 10000
E = 160000
F_IN = 50
HID = 256
H1 = 4
H3 = 6
C_OUT = 121

L = 16
NW = 32
BD = 64
NBLK = (N + BD - 1) // BD
N_PAD = NBLK * BD
RP_LEN = N_PAD + 80
EP = 163840
EPW = EP // NW
EC = 512
W_WIN = 2048

_mesh = functools.partial(
    plsc.VectorSubcoreMesh, core_axis_name="c", subcore_axis_name="s")


def _iota():
    return lax.iota(jnp.int32, L)


def _lane_i(v, j):
    return jnp.sum(jnp.where(_iota() == j, v, 0))


def _lane_f(v, j):
    return jnp.sum(jnp.where(_iota() == j, v, jnp.float32(0.0)))


def _wid():
    return lax.axis_index("s") * 2 + lax.axis_index("c")


def _edge_scores(es, ed, srcs_pad, dsts_pad, hg):
    nvec = EC // L

    def body(es_hbm, ed_hbm, src_hbm, dst_hbm, out_hbm,
             es_t, ed_t, src_c, dst_c, esc_o):
        w = _wid()
        base = w * EPW
        pltpu.sync_copy(es_hbm, es_t)
        pltpu.sync_copy(ed_hbm, ed_t)

        def chunk(c, _):
            off = pl.multiple_of(base + c * EC, 512)
            pltpu.sync_copy(src_hbm.at[pl.ds(off, EC)], src_c)
            pltpu.sync_copy(dst_hbm.at[pl.ds(off, EC)], dst_c)
            for v in range(nvec):
                s = src_c[pl.ds(v * L, L)]
                d = dst_c[pl.ds(v * L, L)]
                dc = jnp.minimum(d, N - 1)
                for k in range(hg):
                    a = plsc.load_gather(es_t, [s * hg + k])
                    b = plsc.load_gather(ed_t, [dc * hg + k])
                    e = a + b
                    ev = jnp.where(e > 0, e, e * jnp.float32(0.2))
                    esc_o[pl.ds(k * EC + v * L, L)] = ev
            for k in range(hg):
                pltpu.sync_copy(esc_o.at[pl.ds(k * EC, EC)],
                                out_hbm.at[pl.ds(k * EP + off, EC)])
            return _

        lax.fori_loop(0, EPW // EC, chunk, 0)

    f = pl.kernel(
        body,
        out_type=jax.ShapeDtypeStruct((hg * EP,), jnp.float32),
        mesh=_mesh(),
        compiler_params=pltpu.CompilerParams(needs_layout_passes=False),
        scratch_types=[
            pltpu.VMEM((N * hg,), jnp.float32),
            pltpu.VMEM((N * hg,), jnp.float32),
            pltpu.VMEM((EC,), jnp.int32),
            pltpu.VMEM((EC,), jnp.int32),
            pltpu.VMEM((hg * EC,), jnp.float32),
        ],
    )
    return f(es.reshape(-1), ed.reshape(-1), srcs_pad, dsts_pad)


def _message(feat, escore, rp_pad, srcs_pad, dsts_pad, h, din, dout, mode):
    ngrp = BD // L
    neg = jnp.float32(-1e30)

    def body(feat_hbm, esc_hbm, rp_hbm, src_hbm, dst_hbm, out_hbm,
             acc, esc_w, src_w, dst_w, rp_b, mx_t, dn_t, rows, sem):
        w = _wid()

        def block(bi, _):
            blk = w + bi * NW
            d0 = pl.multiple_of(blk * BD, BD)

            def z(i, _):
                acc[pl.ds(i * L, L)] = jnp.zeros((L,), jnp.float32)
                return _
            lax.fori_loop(0, BD * dout // L, z, 0)
            for i in range(BD * h // L):
                mx_t[pl.ds(i * L, L)] = jnp.full((L,), neg)
                dn_t[pl.ds(i * L, L)] = jnp.zeros((L,), jnp.float32)

            pltpu.sync_copy(rp_hbm.at[pl.ds(d0, 80)], rp_b)
            e0 = _lane_i(rp_b[pl.ds(0, L)], 0)
            e1 = _lane_i(rp_b[pl.ds(64, L)], 0)
            e0a = pl.multiple_of(e0 & ~7, 8)
            n_win = (e1 - e0a + W_WIN - 1) // W_WIN

            def stage_esc(wi):
                ws = pl.multiple_of(e0a + wi * W_WIN, 8)
                for k in range(h):
                    pltpu.sync_copy(esc_hbm.at[pl.ds(k * EP + ws, W_WIN)],
                                    esc_w.at[pl.ds(k * W_WIN, W_WIN)])
                return ws

            def win1(wi, _):
                ws = stage_esc(wi)
                for g in range(ngrp):
                    lo = plsc.load_gather(rp_b, [g * L + _iota()])
                    hi = plsc.load_gather(rp_b, [g * L + 1 + _iota()])
                    deg = hi - lo
                    mdeg = jnp.max(deg)

                    def jb(j, mxs):
                        local = lo + j - ws
                        m = (j < deg) & (local >= 0) & (local < W_WIN)
                        lidx = jnp.clip(local, 0, W_WIN - 1)
                        out = []
                        for k in range(h):
                            sv = plsc.load_gather(esc_w, [k * W_WIN + lidx])
                            out.append(
                                jnp.maximum(mxs[k], jnp.where(m, sv, neg)))
                        return tuple(out)

                    mxs = lax.fori_loop(0, mdeg, jb,
                                        tuple(jnp.full((L,), neg)
                                              for _ in range(h)))
                    for k in range(h):
                        sl = pl.ds(k * BD + g * L, L)
                        mx_t[sl] = jnp.maximum(mx_t[sl], mxs[k])
                return _
            lax.fori_loop(0, n_win, win1, 0)

            def win2(wi, _):
                ws = stage_esc(wi)
                for g in range(ngrp):
                    lo = plsc.load_gather(rp_b, [g * L + _iota()])
                    hi = plsc.load_gather(rp_b, [g * L + 1 + _iota()])
                    deg = hi - lo
                    mdeg = jnp.max(deg)
                    mxk = [mx_t[pl.ds(k * BD + g * L, L)] for k in range(h)]

                    def jb(j, dns):
                        local = lo + j - ws
                        m = (j < deg) & (local >= 0) & (local < W_WIN)
                        lidx = jnp.clip(local, 0, W_WIN - 1)
                        out = []
                        for k in range(h):
                            sv = plsc.load_gather(esc_w, [k * W_WIN + lidx])
                            ex = jnp.exp(sv - mxk[k])
                            out.append(dns[k] +
                                       jnp.where(m, ex, jnp.float32(0.0)))
                        return tuple(out)

                    dns = lax.fori_loop(0, mdeg, jb,
                                        tuple(jnp.zeros((L,), jnp.float32)
                                              for _ in range(h)))
                    for k in range(h):
                        sl = pl.ds(k * BD + g * L, L)
                        dn_t[sl] = dn_t[sl] + dns[k]
                return _
            lax.fori_loop(0, n_win, win2, 0)

            def win3(wi, _):
                ws = stage_esc(wi)
                pltpu.sync_copy(src_hbm.at[pl.ds(ws, W_WIN)], src_w)
                pltpu.sync_copy(dst_hbm.at[pl.ds(ws, W_WIN)], dst_w)
                n_ch = jnp.clip((e1 - ws + L - 1) // L, 0, W_WIN // L)

                def issue(c):
                    sv_src = src_w[pl.ds(c * L, L)]
                    pltpu.async_copy(feat_hbm.at[sv_src],
                                     rows.at[pl.ds((c & 1) * L, L)], sem)

                @pl.when(n_ch > 0)
                def _prime():
                    issue(0)

                def chunk(c, _):
                    dv = dst_w[pl.ds(c * L, L)]
                    dloc = dv - d0
                    valid = (dloc >= 0) & (dloc < BD)
                    dc = jnp.clip(dloc, 0, BD - 1)
                    als = []
                    for k in range(h):
                        sv = esc_w[pl.ds(k * W_WIN + c * L, L)]
                        mx = plsc.load_gather(mx_t, [k * BD + dc])
                        dn = plsc.load_gather(dn_t, [k * BD + dc])
                        al = jnp.exp(sv - mx) / (dn + jnp.float32(1e-16))
                        als.append(jnp.where(valid, al, jnp.float32(0.0)))

                    @pl.when(c + 1 < n_ch)
                    def _pref():
                        issue(c + 1)

                    pltpu.make_async_copy(
                        feat_hbm.at[src_w[pl.ds(c * L, L)]],
                        rows.at[pl.ds((c & 1) * L, L)], sem).wait()
                    rbase = (c & 1) * L

                    def rrow(je, off):
                        return plsc.load_gather(
                            rows, [jnp.full((L,), rbase, jnp.int32) + je,
                                   off + _iota()])

                    def je_body(je, _):
                        dj = _lane_i(dc, je)
                        obase = dj * dout
                        if mode == 2:
                            for k in range(h):
                                a = _lane_f(als[k], je)
                                for f in range(HID // L):
                                    r = rrow(je, k * HID + f * L)
                                    plsc.addupdate(
                                        acc.at[pl.ds(
                                            obase + k * HID + f * L, L)],
                                        a * r)
                        elif mode == 1:
                            rr = [rrow(je, f * L) for f in range(4)]
                            for k in range(h):
                                a = _lane_f(als[k], je)
                                for f in range(4):
                                    plsc.addupdate(
                                        acc.at[pl.ds(
                                            obase + k * 64 + f * L, L)],
                                        a * rr[f])
                        else:
                            avs = [_lane_f(als[k], je) for k in range(h)]
                            tail = jnp.where(_iota() < 9, jnp.float32(1.0),
                                             jnp.float32(0.0))
                            for f in range(8):
                                s = jnp.zeros((L,), jnp.float32)
                                for k in range(h):
                                    r = rrow(je, k * C_OUT + f * L)
                                    s = s + avs[k] * r
                                if f == 7:
                                    s = s * tail
                                plsc.addupdate(
                                    acc.at[pl.ds(obase + f * L, L)], s)
                        return _

                    lax.fori_loop(0, L, je_body, 0)
                    return _

                lax.fori_loop(0, n_ch, chunk, 0)
                return _
            lax.fori_loop(0, n_win, win3, 0)

            pltpu.sync_copy(acc, out_hbm.at[pl.ds(pl.multiple_of(d0 * dout, 128), BD * dout)])
            return _

        nb = NBLK // NW + jnp.where(w < NBLK % NW, 1, 0)
        lax.fori_loop(0, nb, block, 0)

    f = pl.kernel(
        body,
        out_type=jax.ShapeDtypeStruct((N_PAD * dout,), jnp.float32),
        mesh=_mesh(),
        compiler_params=pltpu.CompilerParams(needs_layout_passes=False),
        scratch_types=[
            pltpu.VMEM((BD * dout,), jnp.float32),
            pltpu.VMEM((h * W_WIN,), jnp.float32),
            pltpu.VMEM((W_WIN,), jnp.int32),
            pltpu.VMEM((W_WIN,), jnp.int32),
            pltpu.VMEM((80,), jnp.int32),
            pltpu.VMEM((h * BD,), jnp.float32),
            pltpu.VMEM((h * BD,), jnp.float32),
            pltpu.VMEM((2 * L, din), jnp.float32),
            pltpu.SemaphoreType.DMA,
        ],
    )
    return f(feat, escore, rp_pad, srcs_pad, dsts_pad)


def _k0(x128, uv1p):
    def body(x_ref, w_ref, o_ref):
        o_ref[...] = jnp.dot(x_ref[...], w_ref[...],
                             preferred_element_type=jnp.float32)
    return pl.pallas_call(
        body,
        grid=(10,),
        in_specs=[pl.BlockSpec((1000, 128), lambda i: (i, 0)),
                  pl.BlockSpec((128, 128), lambda i: (0, 0))],
        out_specs=pl.BlockSpec((1000, 128), lambda i: (i, 0)),
        out_shape=jax.ShapeDtypeStruct((N, 128), jnp.float32),
    )(x128, uv1p)


def _ka(g1p, x128, B1p, Wl1p, bias1, W2, uv2p, Wl2):
    def body(g_ref, x_ref, b1_ref, wl1_ref, bias_ref, w2_ref, uv_ref,
             wl2_ref, h2_ref, sc_ref, sk_ref):
        x1 = jnp.dot(g_ref[...], b1_ref[...],
                     preferred_element_type=jnp.float32)
        x1 = x1 + jnp.dot(x_ref[...], wl1_ref[...],
                          preferred_element_type=jnp.float32)
        x1 = x1 + bias_ref[...]
        x1 = jnp.where(x1 > 0, x1, jnp.exp(jnp.minimum(x1, 0.0)) - 1.0)
        h2_ref[...] = jnp.dot(x1, w2_ref[...],
                              preferred_element_type=jnp.float32)
        sc_ref[...] = jnp.dot(x1, uv_ref[...],
                              preferred_element_type=jnp.float32)
        sk_ref[...] = jnp.dot(x1, wl2_ref[...],
                              preferred_element_type=jnp.float32)

    bn = 1000
    return pl.pallas_call(
        body,
        grid=(N // bn,),
        in_specs=[pl.BlockSpec((bn, 256), lambda i: (i, 0)),
                  pl.BlockSpec((bn, 128), lambda i: (i, 0)),
                  pl.BlockSpec((256, 1024), lambda i: (0, 0)),
                  pl.BlockSpec((128, 1024), lambda i: (0, 0)),
                  pl.BlockSpec((1, 1024), lambda i: (0, 0)),
                  pl.BlockSpec((1024, 1024), lambda i: (0, 0)),
                  pl.BlockSpec((1024, 128), lambda i: (0, 0)),
                  pl.BlockSpec((1024, 1024), lambda i: (0, 0))],
        out_specs=[pl.BlockSpec((bn, 1024), lambda i: (i, 0)),
                   pl.BlockSpec((bn, 128), lambda i: (i, 0)),
                   pl.BlockSpec((bn, 1024), lambda i: (i, 0))],
        out_shape=[jax.ShapeDtypeStruct((N, 1024), jnp.float32),
                   jax.ShapeDtypeStruct((N, 128), jnp.float32),
                   jax.ShapeDtypeStruct((N, 1024), jnp.float32)],
    )(g1p, x128, B1p, Wl1p, bias1, W2, uv2p, Wl2)


def _kb(m2, sk2, bias2, W3p, uv3p, Wl3p):
    def body(m_ref, sk_ref, bias_ref, w3_ref, uv_ref, wl3_ref,
             h3_ref, sc_ref, sk3_ref):
        x2 = m_ref[...] + sk_ref[...] + bias_ref[...]
        x2 = jnp.where(x2 > 0, x2, jnp.exp(jnp.minimum(x2, 0.0)) - 1.0)
        h3_ref[...] = jnp.dot(x2, w3_ref[...],
                              preferred_element_type=jnp.float32)
        sc_ref[...] = jnp.dot(x2, uv_ref[...],
                              preferred_element_type=jnp.float32)
        sk3_ref[...] = jnp.dot(x2, wl3_ref[...],
                               preferred_element_type=jnp.float32)

    bn = 1000
    return pl.pallas_call(
        body,
        grid=(N // bn,),
        in_specs=[pl.BlockSpec((bn, 1024), lambda i: (i, 0)),
                  pl.BlockSpec((bn, 1024), lambda i: (i, 0)),
                  pl.BlockSpec((1, 1024), lambda i: (0, 0)),
                  pl.BlockSpec((1024, 768), lambda i: (0, 0)),
                  pl.BlockSpec((1024, 128), lambda i: (0, 0)),
                  pl.BlockSpec((1024, 128), lambda i: (0, 0))],
        out_specs=[pl.BlockSpec((bn, 768), lambda i: (i, 0)),
                   pl.BlockSpec((bn, 128), lambda i: (i, 0)),
                   pl.BlockSpec((bn, 128), lambda i: (i, 0))],
        out_shape=[jax.ShapeDtypeStruct((N, 768), jnp.float32),
                   jax.ShapeDtypeStruct((N, 128), jnp.float32),
                   jax.ShapeDtypeStruct((N, 128), jnp.float32)],
    )(m2, sk2, bias2, W3p, uv3p, Wl3p)


def _kc(m3, sk3, bias3):
    def body(m_ref, sk_ref, b_ref, o_ref):
        v = (m_ref[...] * jnp.float32(1.0 / H3) + sk_ref[...] + b_ref[...])
        o_ref[...] = v[:, :C_OUT]

    bn = 1000
    return pl.pallas_call(
        body,
        grid=(N // bn,),
        in_specs=[pl.BlockSpec((bn, 128), lambda i: (i, 0)),
                  pl.BlockSpec((bn, 128), lambda i: (i, 0)),
                  pl.BlockSpec((1, 128), lambda i: (0, 0))],
        out_specs=pl.BlockSpec((bn, C_OUT), lambda i: (i, 0)),
        out_shape=jax.ShapeDtypeStruct((N, C_OUT), jnp.float32),
    )(m3, sk3, bias3)


def kernel(x, edge_index, W1, a_src1, a_dst1, b1, Wl1, bl1, W2, a_src2,
           a_dst2, b2, Wl2, bl2, W3, a_src3, a_dst3, b3, Wl3, bl3):
    src = edge_index[0]
    dst = edge_index[1]

    perm = jnp.argsort(dst)
    srcs = src[perm]
    dsts = dst[perm]
    srcs_pad = jnp.zeros((EP,), jnp.int32).at[:E].set(srcs)
    dsts_pad = jnp.full((EP,), 2 * N_PAD, jnp.int32).at[:E].set(dsts)
    rp_pad = jnp.searchsorted(dsts, jnp.arange(RP_LEN),
                              side="left").astype(jnp.int32)

    def fold(W, a_s, a_d, heads, ch):
        Wr = W.reshape(W.shape[0], heads, ch)
        u = jnp.einsum("fkc,kc->fk", Wr, a_s)
        v = jnp.einsum("fkc,kc->fk", Wr, a_d)
        return jnp.concatenate([u, v], axis=1)

    uv1 = fold(W1, a_src1, a_dst1, H1, HID)
    uv2 = fold(W2, a_src2, a_dst2, H1, HID)
    uv3 = fold(W3, a_src3, a_dst3, H3, C_OUT)
    uv1p = jnp.zeros((128, 128), jnp.float32).at[:F_IN, :2 * H1].set(uv1)
    uv2p = jnp.zeros((1024, 128), jnp.float32).at[:, :2 * H1].set(uv2)
    uv3p = jnp.zeros((1024, 128), jnp.float32).at[:, :2 * H3].set(uv3)

    x128 = jnp.zeros((N, 128), jnp.float32).at[:, :F_IN].set(x)
    B1p = jnp.zeros((256, 1024), jnp.float32)
    W1r = W1.reshape(F_IN, H1, HID)
    for k in range(H1):
        B1p = B1p.at[k * 64:k * 64 + F_IN,
                     k * HID:(k + 1) * HID].set(W1r[:, k, :])
    Wl1p = jnp.zeros((128, 1024), jnp.float32).at[:F_IN, :].set(Wl1)
    bias1 = (b1 + bl1).reshape(1, 1024)
    bias2 = (b2 + bl2).reshape(1, 1024)
    W3p = jnp.zeros((1024, 768), jnp.float32).at[:, :H3 * C_OUT].set(W3)
    Wl3p = jnp.zeros((1024, 128), jnp.float32).at[:, :C_OUT].set(Wl3)
    bias3 = jnp.zeros((1, 128), jnp.float32).at[0, :C_OUT].set(b3 + bl3)

    sc1 = _k0(x128, uv1p)
    esc1 = _edge_scores(sc1[:, :H1], sc1[:, H1:2 * H1], srcs_pad, dsts_pad,
                        H1)
    g1 = _message(x128, esc1, rp_pad, srcs_pad, dsts_pad, H1, 128, 256, 1)
    g1p = g1[:N * 256].reshape(N, 256)

    h2, sc2, sk2 = _ka(g1p, x128, B1p, Wl1p, bias1, W2, uv2p, Wl2)
    esc2 = _edge_scores(sc2[:, :H1], sc2[:, H1:2 * H1], srcs_pad, dsts_pad,
                        H1)
    m2 = _message(h2, esc2, rp_pad, srcs_pad, dsts_pad, H1, 1024, 1024, 2)
    m2 = m2[:N * 1024].reshape(N, 1024)

    h3, sc3, sk3 = _kb(m2, sk2, bias2, W3p, uv3p, Wl3p)
    esc3a = _edge_scores(sc3[:, :3], sc3[:, H3:H3 + 3], srcs_pad, dsts_pad, 3)
    esc3b = _edge_scores(sc3[:, 3:H3], sc3[:, H3 + 3:2 * H3], srcs_pad,
                         dsts_pad, 3)
    esc3 = jnp.concatenate([esc3a, esc3b], axis=0)
    m3 = _message(h3, esc3, rp_pad, srcs_pad, dsts_pad, H3, 768, 128, 3)
    m3 = m3[:N * 128].reshape(N, 128)

    return _kc(m3, sk3, bias3)


# speedup vs baseline: 17.5221x; 1.0455x over previous
import functools

import jax
import jax.numpy as jnp
from jax import lax
from jax.experimental import pallas as pl
from jax.experimental.pallas import tpu as pltpu
from jax.experimental.pallas import tpu_sc as plsc

N = 10000
E = 160000
F_IN = 50
HID = 256
H1 = 4
H3 = 6
C_OUT = 121

L = 16
NW = 32
BD = 64
NBLK = (N + BD - 1) // BD
N_PAD = NBLK * BD
RP_LEN = N_PAD + 80
EP = 163840
EPW = EP // NW
EC = 512
W_WIN = 2048

_mesh = functools.partial(
    plsc.VectorSubcoreMesh, core_axis_name="c", subcore_axis_name="s")


def _iota():
    return lax.iota(jnp.int32, L)


def _lane_i(v, j):
    return jnp.sum(jnp.where(_iota() == j, v, 0))


def _lane_f(v, j):
    return jnp.sum(jnp.where(_iota() == j, v, jnp.float32(0.0)))


def _wid():
    return lax.axis_index("s") * 2 + lax.axis_index("c")


def _edge_scores(es, ed, srcs_pad, dsts_pad, hg):
    nvec = EC // L

    def body(es_hbm, ed_hbm, src_hbm, dst_hbm, out_hbm,
             es_t, ed_t, src_c, dst_c, esc_o):
        w = _wid()
        base = w * EPW
        pltpu.sync_copy(es_hbm, es_t)
        pltpu.sync_copy(ed_hbm, ed_t)

        def chunk(c, _):
            off = pl.multiple_of(base + c * EC, 512)
            pltpu.sync_copy(src_hbm.at[pl.ds(off, EC)], src_c)
            pltpu.sync_copy(dst_hbm.at[pl.ds(off, EC)], dst_c)
            for v in range(nvec):
                s = src_c[pl.ds(v * L, L)]
                d = dst_c[pl.ds(v * L, L)]
                dc = jnp.minimum(d, N - 1)
                for k in range(hg):
                    a = plsc.load_gather(es_t, [s * hg + k])
                    b = plsc.load_gather(ed_t, [dc * hg + k])
                    e = a + b
                    ev = jnp.where(e > 0, e, e * jnp.float32(0.2))
                    esc_o[pl.ds(k * EC + v * L, L)] = ev
            for k in range(hg):
                pltpu.sync_copy(esc_o.at[pl.ds(k * EC, EC)],
                                out_hbm.at[pl.ds(k * EP + off, EC)])
            return _

        lax.fori_loop(0, EPW // EC, chunk, 0)

    f = pl.kernel(
        body,
        out_type=jax.ShapeDtypeStruct((hg * EP,), jnp.float32),
        mesh=_mesh(),
        compiler_params=pltpu.CompilerParams(needs_layout_passes=False),
        scratch_types=[
            pltpu.VMEM((N * hg,), jnp.float32),
            pltpu.VMEM((N * hg,), jnp.float32),
            pltpu.VMEM((EC,), jnp.int32),
            pltpu.VMEM((EC,), jnp.int32),
            pltpu.VMEM((hg * EC,), jnp.float32),
        ],
    )
    return f(es.reshape(-1), ed.reshape(-1), srcs_pad, dsts_pad)


def _message(feat, escore, rp_pad, srcs_pad, dsts_pad, h, din, dout, mode):
    ngrp = BD // L
    neg = jnp.float32(-1e30)
    ring = 2 if mode == 2 else 4

    def body(feat_hbm, esc_hbm, rp_hbm, src_hbm, dst_hbm, out_hbm,
             acc, esc_w, src_w, dst_w, rp_b, mx_t, dn_t, rows, sem):
        w = _wid()

        def block(bi, _):
            blk = w + bi * NW
            d0 = pl.multiple_of(blk * BD, BD)

            def z(i, _):
                acc[pl.ds(i * L, L)] = jnp.zeros((L,), jnp.float32)
                return _
            lax.fori_loop(0, BD * dout // L, z, 0)
            for i in range(BD * h // L):
                mx_t[pl.ds(i * L, L)] = jnp.full((L,), neg)
                dn_t[pl.ds(i * L, L)] = jnp.zeros((L,), jnp.float32)

            pltpu.sync_copy(rp_hbm.at[pl.ds(d0, 80)], rp_b)
            e0 = _lane_i(rp_b[pl.ds(0, L)], 0)
            e1 = _lane_i(rp_b[pl.ds(64, L)], 0)
            e0a = pl.multiple_of(e0 & ~7, 8)
            n_win = (e1 - e0a + W_WIN - 1) // W_WIN

            def stage_esc(wi):
                ws = pl.multiple_of(e0a + wi * W_WIN, 8)
                for k in range(h):
                    pltpu.sync_copy(esc_hbm.at[pl.ds(k * EP + ws, W_WIN)],
                                    esc_w.at[pl.ds(k * W_WIN, W_WIN)])
                return ws

            def win12(wi, _):
                ws = stage_esc(wi)
                for g in range(ngrp):
                    lo = plsc.load_gather(rp_b, [g * L + _iota()])
                    hi = plsc.load_gather(rp_b, [g * L + 1 + _iota()])
                    deg = hi - lo
                    mdeg = jnp.max(deg)

                    def jb(j, carry):
                        mxs = carry[:h]
                        dns = carry[h:]
                        local = lo + j - ws
                        m = (j < deg) & (local >= 0) & (local < W_WIN)
                        lidx = jnp.clip(local, 0, W_WIN - 1)
                        nm = []
                        nd = []
                        for k in range(h):
                            sv = plsc.load_gather(esc_w, [k * W_WIN + lidx])
                            svm = jnp.where(m, sv, neg)
                            mk = jnp.maximum(mxs[k], svm)
                            nd.append(dns[k] * jnp.exp(mxs[k] - mk)
                                      + jnp.where(m, jnp.exp(sv - mk),
                                                  jnp.float32(0.0)))
                            nm.append(mk)
                        return tuple(nm) + tuple(nd)

                    init = (tuple(jnp.full((L,), neg) for _ in range(h))
                            + tuple(jnp.zeros((L,), jnp.float32)
                                    for _ in range(h)))
                    res = lax.fori_loop(0, mdeg, jb, init)
                    for k in range(h):
                        sl = pl.ds(k * BD + g * L, L)
                        om = mx_t[sl]
                        nm2 = jnp.maximum(om, res[k])
                        dn_t[sl] = (dn_t[sl] * jnp.exp(om - nm2)
                                    + res[h + k] * jnp.exp(res[k] - nm2))
                        mx_t[sl] = nm2
                return _
            lax.fori_loop(0, n_win, win12, 0)

            def win3(wi, _):
                ws = pl.multiple_of(e0a + wi * W_WIN, 8)

                @pl.when(n_win > 1)
                def _restage():
                    stage_esc(wi)
                pltpu.sync_copy(src_hbm.at[pl.ds(ws, W_WIN)], src_w)
                pltpu.sync_copy(dst_hbm.at[pl.ds(ws, W_WIN)], dst_w)
                n_ch = jnp.clip((e1 - ws + L - 1) // L, 0, W_WIN // L)

                def issue(c):
                    sv_src = src_w[pl.ds(c * L, L)]
                    pltpu.async_copy(feat_hbm.at[sv_src],
                                     rows.at[pl.ds((c % ring) * L, L)], sem)

                for p in range(ring - 1):
                    @pl.when(p < n_ch)
                    def _prime(p=p):
                        issue(p)

                def chunk(c, _):
                    dv = dst_w[pl.ds(c * L, L)]
                    dloc = dv - d0
                    valid = (dloc >= 0) & (dloc < BD)
                    dc = jnp.clip(dloc, 0, BD - 1)
                    als = []
                    for k in range(h):
                        sv = esc_w[pl.ds(k * W_WIN + c * L, L)]
                        mx = plsc.load_gather(mx_t, [k * BD + dc])
                        dn = plsc.load_gather(dn_t, [k * BD + dc])
                        al = jnp.exp(sv - mx) / (dn + jnp.float32(1e-16))
                        als.append(jnp.where(valid, al, jnp.float32(0.0)))

                    @pl.when(c + ring - 1 < n_ch)
                    def _pref():
                        issue(c + ring - 1)

                    pltpu.make_async_copy(
                        feat_hbm.at[src_w[pl.ds(c * L, L)]],
                        rows.at[pl.ds((c % ring) * L, L)], sem).wait()
                    rbase = (c % ring) * L

                    def rrow(je, off):
                        return plsc.load_gather(
                            rows, [jnp.full((L,), rbase, jnp.int32) + je,
                                   off + _iota()])

                    def je_body(je, _):
                        dj = _lane_i(dc, je)
                        obase = dj * dout
                        if mode == 2:
                            for k in range(h):
                                a = _lane_f(als[k], je)
                                for f in range(HID // L):
                                    r = rrow(je, k * HID + f * L)
                                    plsc.addupdate(
                                        acc.at[pl.ds(
                                            obase + k * HID + f * L, L)],
                                        a * r)
                        elif mode == 1:
                            rr = [rrow(je, f * L) for f in range(4)]
                            for k in range(h):
                                a = _lane_f(als[k], je)
                                for f in range(4):
                                    plsc.addupdate(
                                        acc.at[pl.ds(
                                            obase + k * 64 + f * L, L)],
                                        a * rr[f])
                        else:
                            avs = [_lane_f(als[k], je) for k in range(h)]
                            tail = jnp.where(_iota() < 9, jnp.float32(1.0),
                                             jnp.float32(0.0))
                            for f in range(8):
                                s = jnp.zeros((L,), jnp.float32)
                                for k in range(h):
                                    r = rrow(je, k * C_OUT + f * L)
                                    s = s + avs[k] * r
                                if f == 7:
                                    s = s * tail
                                plsc.addupdate(
                                    acc.at[pl.ds(obase + f * L, L)], s)
                        return _

                    lax.fori_loop(0, L, je_body, 0)
                    return _

                lax.fori_loop(0, n_ch, chunk, 0)
                return _
            lax.fori_loop(0, n_win, win3, 0)

            pltpu.sync_copy(acc, out_hbm.at[pl.ds(pl.multiple_of(d0 * dout, 128), BD * dout)])
            return _

        nb = NBLK // NW + jnp.where(w < NBLK % NW, 1, 0)
        lax.fori_loop(0, nb, block, 0)

    f = pl.kernel(
        body,
        out_type=jax.ShapeDtypeStruct((N_PAD * dout,), jnp.float32),
        mesh=_mesh(),
        compiler_params=pltpu.CompilerParams(needs_layout_passes=False),
        scratch_types=[
            pltpu.VMEM((BD * dout,), jnp.float32),
            pltpu.VMEM((h * W_WIN,), jnp.float32),
            pltpu.VMEM((W_WIN,), jnp.int32),
            pltpu.VMEM((W_WIN,), jnp.int32),
            pltpu.VMEM((80,), jnp.int32),
            pltpu.VMEM((h * BD,), jnp.float32),
            pltpu.VMEM((h * BD,), jnp.float32),
            pltpu.VMEM((ring * L, din), jnp.float32),
            pltpu.SemaphoreType.DMA,
        ],
    )
    return f(feat, escore, rp_pad, srcs_pad, dsts_pad)


def _k0(x128, uv1p):
    def body(x_ref, w_ref, o_ref):
        o_ref[...] = jnp.dot(x_ref[...], w_ref[...],
                             preferred_element_type=jnp.float32)
    return pl.pallas_call(
        body,
        grid=(10,),
        in_specs=[pl.BlockSpec((1000, 128), lambda i: (i, 0)),
                  pl.BlockSpec((128, 128), lambda i: (0, 0))],
        out_specs=pl.BlockSpec((1000, 128), lambda i: (i, 0)),
        out_shape=jax.ShapeDtypeStruct((N, 128), jnp.float32),
    )(x128, uv1p)


def _ka(g1p, x128, B1p, Wl1p, bias1, W2, uv2p, Wl2):
    def body(g_ref, x_ref, b1_ref, wl1_ref, bias_ref, w2_ref, uv_ref,
             wl2_ref, h2_ref, sc_ref, sk_ref):
        x1 = jnp.dot(g_ref[...], b1_ref[...],
                     preferred_element_type=jnp.float32)
        x1 = x1 + jnp.dot(x_ref[...], wl1_ref[...],
                          preferred_element_type=jnp.float32)
        x1 = x1 + bias_ref[...]
        x1 = jnp.where(x1 > 0, x1, jnp.exp(jnp.minimum(x1, 0.0)) - 1.0)
        h2_ref[...] = jnp.dot(x1, w2_ref[...],
                              preferred_element_type=jnp.float32)
        sc_ref[...] = jnp.dot(x1, uv_ref[...],
                              preferred_element_type=jnp.float32)
        sk_ref[...] = jnp.dot(x1, wl2_ref[...],
                              preferred_element_type=jnp.float32)

    bn = 1000
    return pl.pallas_call(
        body,
        grid=(N // bn,),
        in_specs=[pl.BlockSpec((bn, 256), lambda i: (i, 0)),
                  pl.BlockSpec((bn, 128), lambda i: (i, 0)),
                  pl.BlockSpec((256, 1024), lambda i: (0, 0)),
                  pl.BlockSpec((128, 1024), lambda i: (0, 0)),
                  pl.BlockSpec((1, 1024), lambda i: (0, 0)),
                  pl.BlockSpec((1024, 1024), lambda i: (0, 0)),
                  pl.BlockSpec((1024, 128), lambda i: (0, 0)),
                  pl.BlockSpec((1024, 1024), lambda i: (0, 0))],
        out_specs=[pl.BlockSpec((bn, 1024), lambda i: (i, 0)),
                   pl.BlockSpec((bn, 128), lambda i: (i, 0)),
                   pl.BlockSpec((bn, 1024), lambda i: (i, 0))],
        out_shape=[jax.ShapeDtypeStruct((N, 1024), jnp.float32),
                   jax.ShapeDtypeStruct((N, 128), jnp.float32),
                   jax.ShapeDtypeStruct((N, 1024), jnp.float32)],
    )(g1p, x128, B1p, Wl1p, bias1, W2, uv2p, Wl2)


def _kb(m2, sk2, bias2, W3p, uv3p, Wl3p):
    def body(m_ref, sk_ref, bias_ref, w3_ref, uv_ref, wl3_ref,
             h3_ref, sc_ref, sk3_ref):
        x2 = m_ref[...] + sk_ref[...] + bias_ref[...]
        x2 = jnp.where(x2 > 0, x2, jnp.exp(jnp.minimum(x2, 0.0)) - 1.0)
        h3_ref[...] = jnp.dot(x2, w3_ref[...],
                              preferred_element_type=jnp.float32)
        sc_ref[...] = jnp.dot(x2, uv_ref[...],
                              preferred_element_type=jnp.float32)
        sk3_ref[...] = jnp.dot(x2, wl3_ref[...],
                               preferred_element_type=jnp.float32)

    bn = 1000
    return pl.pallas_call(
        body,
        grid=(N // bn,),
        in_specs=[pl.BlockSpec((bn, 1024), lambda i: (i, 0)),
                  pl.BlockSpec((bn, 1024), lambda i: (i, 0)),
                  pl.BlockSpec((1, 1024), lambda i: (0, 0)),
                  pl.BlockSpec((1024, 768), lambda i: (0, 0)),
                  pl.BlockSpec((1024, 128), lambda i: (0, 0)),
                  pl.BlockSpec((1024, 128), lambda i: (0, 0))],
        out_specs=[pl.BlockSpec((bn, 768), lambda i: (i, 0)),
                   pl.BlockSpec((bn, 128), lambda i: (i, 0)),
                   pl.BlockSpec((bn, 128), lambda i: (i, 0))],
        out_shape=[jax.ShapeDtypeStruct((N, 768), jnp.float32),
                   jax.ShapeDtypeStruct((N, 128), jnp.float32),
                   jax.ShapeDtypeStruct((N, 128), jnp.float32)],
    )(m2, sk2, bias2, W3p, uv3p, Wl3p)


def _kc(m3, sk3, bias3):
    def body(m_ref, sk_ref, b_ref, o_ref):
        v = (m_ref[...] * jnp.float32(1.0 / H3) + sk_ref[...] + b_ref[...])
        o_ref[...] = v[:, :C_OUT]

    bn = 1000
    return pl.pallas_call(
        body,
        grid=(N // bn,),
        in_specs=[pl.BlockSpec((bn, 128), lambda i: (i, 0)),
                  pl.BlockSpec((bn, 128), lambda i: (i, 0)),
                  pl.BlockSpec((1, 128), lambda i: (0, 0))],
        out_specs=pl.BlockSpec((bn, C_OUT), lambda i: (i, 0)),
        out_shape=jax.ShapeDtypeStruct((N, C_OUT), jnp.float32),
    )(m3, sk3, bias3)


def kernel(x, edge_index, W1, a_src1, a_dst1, b1, Wl1, bl1, W2, a_src2,
           a_dst2, b2, Wl2, bl2, W3, a_src3, a_dst3, b3, Wl3, bl3):
    src = edge_index[0]
    dst = edge_index[1]

    perm = jnp.argsort(dst)
    srcs = src[perm]
    dsts = dst[perm]
    srcs_pad = jnp.zeros((EP,), jnp.int32).at[:E].set(srcs)
    dsts_pad = jnp.full((EP,), 2 * N_PAD, jnp.int32).at[:E].set(dsts)
    rp_pad = jnp.searchsorted(dsts, jnp.arange(RP_LEN),
                              side="left").astype(jnp.int32)

    def fold(W, a_s, a_d, heads, ch):
        Wr = W.reshape(W.shape[0], heads, ch)
        u = jnp.einsum("fkc,kc->fk", Wr, a_s)
        v = jnp.einsum("fkc,kc->fk", Wr, a_d)
        return jnp.concatenate([u, v], axis=1)

    uv1 = fold(W1, a_src1, a_dst1, H1, HID)
    uv2 = fold(W2, a_src2, a_dst2, H1, HID)
    uv3 = fold(W3, a_src3, a_dst3, H3, C_OUT)
    uv1p = jnp.zeros((128, 128), jnp.float32).at[:F_IN, :2 * H1].set(uv1)
    uv2p = jnp.zeros((1024, 128), jnp.float32).at[:, :2 * H1].set(uv2)
    uv3p = jnp.zeros((1024, 128), jnp.float32).at[:, :2 * H3].set(uv3)

    x128 = jnp.zeros((N, 128), jnp.float32).at[:, :F_IN].set(x)
    B1p = jnp.zeros((256, 1024), jnp.float32)
    W1r = W1.reshape(F_IN, H1, HID)
    for k in range(H1):
        B1p = B1p.at[k * 64:k * 64 + F_IN,
                     k * HID:(k + 1) * HID].set(W1r[:, k, :])
    Wl1p = jnp.zeros((128, 1024), jnp.float32).at[:F_IN, :].set(Wl1)
    bias1 = (b1 + bl1).reshape(1, 1024)
    bias2 = (b2 + bl2).reshape(1, 1024)
    W3p = jnp.zeros((1024, 768), jnp.float32).at[:, :H3 * C_OUT].set(W3)
    Wl3p = jnp.zeros((1024, 128), jnp.float32).at[:, :C_OUT].set(Wl3)
    bias3 = jnp.zeros((1, 128), jnp.float32).at[0, :C_OUT].set(b3 + bl3)

    sc1 = _k0(x128, uv1p)
    esc1 = _edge_scores(sc1[:, :H1], sc1[:, H1:2 * H1], srcs_pad, dsts_pad,
                        H1)
    g1 = _message(x128, esc1, rp_pad, srcs_pad, dsts_pad, H1, 128, 256, 1)
    g1p = g1[:N * 256].reshape(N, 256)

    h2, sc2, sk2 = _ka(g1p, x128, B1p, Wl1p, bias1, W2, uv2p, Wl2)
    esc2 = _edge_scores(sc2[:, :H1], sc2[:, H1:2 * H1], srcs_pad, dsts_pad,
                        H1)
    m2 = _message(h2, esc2, rp_pad, srcs_pad, dsts_pad, H1, 1024, 1024, 2)
    m2 = m2[:N * 1024].reshape(N, 1024)

    h3, sc3, sk3 = _kb(m2, sk2, bias2, W3p, uv3p, Wl3p)
    esc3a = _edge_scores(sc3[:, :3], sc3[:, H3:H3 + 3], srcs_pad, dsts_pad, 3)
    esc3b = _edge_scores(sc3[:, 3:H3], sc3[:, H3 + 3:2 * H3], srcs_pad,
                         dsts_pad, 3)
    esc3 = jnp.concatenate([esc3a, esc3b], axis=0)
    m3 = _message(h3, esc3, rp_pad, srcs_pad, dsts_pad, H3, 768, 128, 3)
    m3 = m3[:N * 128].reshape(N, 128)

    return _kc(m3, sk3, bias3)


# speedup vs baseline: 17.8519x; 1.0188x over previous
import functools

import jax
import jax.numpy as jnp
from jax import lax
from jax.experimental import pallas as pl
from jax.experimental.pallas import tpu as pltpu
from jax.experimental.pallas import tpu_sc as plsc

N = 10000
E = 160000
F_IN = 50
HID = 256
H1 = 4
H3 = 6
C_OUT = 121

L = 16
NW = 32
BD = 64
NBLK = (N + BD - 1) // BD
N_PAD = NBLK * BD
RP_LEN = N_PAD + 80
EP = 163840
EPW = EP // NW
EC = 512
W_WIN = 2048

_mesh = functools.partial(
    plsc.VectorSubcoreMesh, core_axis_name="c", subcore_axis_name="s")


def _iota():
    return lax.iota(jnp.int32, L)


def _lane_i(v, j):
    return jnp.sum(jnp.where(_iota() == j, v, 0))


def _lane_f(v, j):
    return jnp.sum(jnp.where(_iota() == j, v, jnp.float32(0.0)))


def _wid():
    return lax.axis_index("s") * 2 + lax.axis_index("c")


def _edge_scores(es, ed, srcs_pad, dsts_pad, hg):
    nvec = EC // L

    def body(es_hbm, ed_hbm, src_hbm, dst_hbm, out_hbm,
             es_t, ed_t, src_c, dst_c, esc_o):
        w = _wid()
        base = w * EPW
        pltpu.sync_copy(es_hbm, es_t)
        pltpu.sync_copy(ed_hbm, ed_t)

        def chunk(c, _):
            off = pl.multiple_of(base + c * EC, 512)
            pltpu.sync_copy(src_hbm.at[pl.ds(off, EC)], src_c)
            pltpu.sync_copy(dst_hbm.at[pl.ds(off, EC)], dst_c)
            for v in range(nvec):
                s = src_c[pl.ds(v * L, L)]
                d = dst_c[pl.ds(v * L, L)]
                dc = jnp.minimum(d, N - 1)
                for k in range(hg):
                    a = plsc.load_gather(es_t, [s * hg + k])
                    b = plsc.load_gather(ed_t, [dc * hg + k])
                    e = a + b
                    ev = jnp.where(e > 0, e, e * jnp.float32(0.2))
                    esc_o[pl.ds(k * EC + v * L, L)] = ev
            for k in range(hg):
                pltpu.sync_copy(esc_o.at[pl.ds(k * EC, EC)],
                                out_hbm.at[pl.ds(k * EP + off, EC)])
            return _

        lax.fori_loop(0, EPW // EC, chunk, 0)

    f = pl.kernel(
        body,
        out_type=jax.ShapeDtypeStruct((hg * EP,), jnp.float32),
        mesh=_mesh(),
        compiler_params=pltpu.CompilerParams(needs_layout_passes=False),
        scratch_types=[
            pltpu.VMEM((N * hg,), jnp.float32),
            pltpu.VMEM((N * hg,), jnp.float32),
            pltpu.VMEM((EC,), jnp.int32),
            pltpu.VMEM((EC,), jnp.int32),
            pltpu.VMEM((hg * EC,), jnp.float32),
        ],
    )
    return f(es.reshape(-1), ed.reshape(-1), srcs_pad, dsts_pad)


def _message(feat, escore, rp_pad, srcs_pad, dsts_pad, h, din, dout, mode):
    ngrp = BD // L
    neg = jnp.float32(-1e30)
    ring = 2 if mode == 2 else 4

    def body(feat_hbm, esc_hbm, rp_hbm, src_hbm, dst_hbm, out_hbm,
             acc, esc_w, src_w, dst_w, rp_b, mx_t, dn_t, rows, sem):
        w = _wid()

        def block(bi, _):
            blk = w + bi * NW
            d0 = pl.multiple_of(blk * BD, BD)

            def z(i, _):
                acc[pl.ds(i * L, L)] = jnp.zeros((L,), jnp.float32)
                return _
            lax.fori_loop(0, BD * dout // L, z, 0)
            for i in range(BD * h // L):
                mx_t[pl.ds(i * L, L)] = jnp.full((L,), neg)
                dn_t[pl.ds(i * L, L)] = jnp.zeros((L,), jnp.float32)

            pltpu.sync_copy(rp_hbm.at[pl.ds(d0, 80)], rp_b)
            e0 = _lane_i(rp_b[pl.ds(0, L)], 0)
            e1 = _lane_i(rp_b[pl.ds(64, L)], 0)
            e0a = pl.multiple_of(e0 & ~7, 8)
            n_win = (e1 - e0a + W_WIN - 1) // W_WIN

            def stage_esc(wi):
                ws = pl.multiple_of(e0a + wi * W_WIN, 8)
                for k in range(h):
                    pltpu.sync_copy(esc_hbm.at[pl.ds(k * EP + ws, W_WIN)],
                                    esc_w.at[pl.ds(k * W_WIN, W_WIN)])
                return ws

            def win12(wi, _):
                ws = stage_esc(wi)
                for g in range(ngrp):
                    lo = plsc.load_gather(rp_b, [g * L + _iota()])
                    hi = plsc.load_gather(rp_b, [g * L + 1 + _iota()])
                    deg = hi - lo
                    mdeg = jnp.max(deg)

                    def jb(j, carry):
                        mxs = carry[:h]
                        dns = carry[h:]
                        local = lo + j - ws
                        m = (j < deg) & (local >= 0) & (local < W_WIN)
                        lidx = jnp.clip(local, 0, W_WIN - 1)
                        nm = []
                        nd = []
                        for k in range(h):
                            sv = plsc.load_gather(esc_w, [k * W_WIN + lidx])
                            svm = jnp.where(m, sv, neg)
                            mk = jnp.maximum(mxs[k], svm)
                            nd.append(dns[k] * jnp.exp(mxs[k] - mk)
                                      + jnp.where(m, jnp.exp(sv - mk),
                                                  jnp.float32(0.0)))
                            nm.append(mk)
                        return tuple(nm) + tuple(nd)

                    init = (tuple(jnp.full((L,), neg) for _ in range(h))
                            + tuple(jnp.zeros((L,), jnp.float32)
                                    for _ in range(h)))
                    res = lax.fori_loop(0, mdeg, jb, init)
                    for k in range(h):
                        sl = pl.ds(k * BD + g * L, L)
                        om = mx_t[sl]
                        nm2 = jnp.maximum(om, res[k])
                        dn_t[sl] = (dn_t[sl] * jnp.exp(om - nm2)
                                    + res[h + k] * jnp.exp(res[k] - nm2))
                        mx_t[sl] = nm2
                return _
            lax.fori_loop(0, n_win, win12, 0)

            def win3(wi, _):
                ws = pl.multiple_of(e0a + wi * W_WIN, 8)

                @pl.when(n_win > 1)
                def _restage():
                    stage_esc(wi)
                pltpu.sync_copy(src_hbm.at[pl.ds(ws, W_WIN)], src_w)
                pltpu.sync_copy(dst_hbm.at[pl.ds(ws, W_WIN)], dst_w)
                n_ch = jnp.clip((e1 - ws + L - 1) // L, 0, W_WIN // L)

                def issue(c):
                    sv_src = src_w[pl.ds(c * L, L)]
                    pltpu.async_copy(feat_hbm.at[sv_src],
                                     rows.at[pl.ds((c % ring) * L, L)], sem)

                for p in range(ring - 1):
                    @pl.when(p < n_ch)
                    def _prime(p=p):
                        issue(p)

                def chunk(c, _):
                    dv = dst_w[pl.ds(c * L, L)]
                    dloc = dv - d0
                    valid = (dloc >= 0) & (dloc < BD)
                    dc = jnp.clip(dloc, 0, BD - 1)
                    als = []
                    for k in range(h):
                        sv = esc_w[pl.ds(k * W_WIN + c * L, L)]
                        mx = plsc.load_gather(mx_t, [k * BD + dc])
                        dn = plsc.load_gather(dn_t, [k * BD + dc])
                        al = jnp.exp(sv - mx) / (dn + jnp.float32(1e-16))
                        als.append(jnp.where(valid, al, jnp.float32(0.0)))

                    @pl.when(c + ring - 1 < n_ch)
                    def _pref():
                        issue(c + ring - 1)

                    pltpu.make_async_copy(
                        feat_hbm.at[src_w[pl.ds(c * L, L)]],
                        rows.at[pl.ds((c % ring) * L, L)], sem).wait()
                    rbase = (c % ring) * L

                    def rrow(je, off):
                        return rows[rbase + je, pl.ds(off, L)]

                    def je_body(je, _):
                        dj = _lane_i(dc, je)
                        obase = dj * dout
                        if mode == 2:
                            for k in range(h):
                                a = _lane_f(als[k], je)
                                for f in range(HID // L):
                                    r = rrow(je, k * HID + f * L)
                                    plsc.addupdate(
                                        acc.at[pl.ds(
                                            obase + k * HID + f * L, L)],
                                        a * r)
                        elif mode == 1:
                            rr = [rrow(je, f * L) for f in range(4)]
                            for k in range(h):
                                a = _lane_f(als[k], je)
                                for f in range(4):
                                    plsc.addupdate(
                                        acc.at[pl.ds(
                                            obase + k * 64 + f * L, L)],
                                        a * rr[f])
                        else:
                            avs = [_lane_f(als[k], je) for k in range(h)]
                            tail = jnp.where(_iota() < 9, jnp.float32(1.0),
                                             jnp.float32(0.0))
                            for f in range(8):
                                s = jnp.zeros((L,), jnp.float32)
                                for k in range(h):
                                    r = rrow(je, k * C_OUT + f * L)
                                    s = s + avs[k] * r
                                if f == 7:
                                    s = s * tail
                                plsc.addupdate(
                                    acc.at[pl.ds(obase + f * L, L)], s)
                        return _

                    lax.fori_loop(0, L, je_body, 0)
                    return _

                lax.fori_loop(0, n_ch, chunk, 0)
                return _
            lax.fori_loop(0, n_win, win3, 0)

            pltpu.sync_copy(acc, out_hbm.at[pl.ds(pl.multiple_of(d0 * dout, 128), BD * dout)])
            return _

        nb = NBLK // NW + jnp.where(w < NBLK % NW, 1, 0)
        lax.fori_loop(0, nb, block, 0)

    f = pl.kernel(
        body,
        out_type=jax.ShapeDtypeStruct((N_PAD * dout,), jnp.float32),
        mesh=_mesh(),
        compiler_params=pltpu.CompilerParams(needs_layout_passes=False),
        scratch_types=[
            pltpu.VMEM((BD * dout,), jnp.float32),
            pltpu.VMEM((h * W_WIN,), jnp.float32),
            pltpu.VMEM((W_WIN,), jnp.int32),
            pltpu.VMEM((W_WIN,), jnp.int32),
            pltpu.VMEM((80,), jnp.int32),
            pltpu.VMEM((h * BD,), jnp.float32),
            pltpu.VMEM((h * BD,), jnp.float32),
            pltpu.VMEM((ring * L, din), jnp.float32),
            pltpu.SemaphoreType.DMA,
        ],
    )
    return f(feat, escore, rp_pad, srcs_pad, dsts_pad)


def _k0(x128, uv1p):
    def body(x_ref, w_ref, o_ref):
        o_ref[...] = jnp.dot(x_ref[...], w_ref[...],
                             preferred_element_type=jnp.float32)
    return pl.pallas_call(
        body,
        grid=(10,),
        in_specs=[pl.BlockSpec((1000, 128), lambda i: (i, 0)),
                  pl.BlockSpec((128, 128), lambda i: (0, 0))],
        out_specs=pl.BlockSpec((1000, 128), lambda i: (i, 0)),
        out_shape=jax.ShapeDtypeStruct((N, 128), jnp.float32),
    )(x128, uv1p)


def _ka(g1p, x128, B1p, Wl1p, bias1, W2, uv2p, Wl2):
    def body(g_ref, x_ref, b1_ref, wl1_ref, bias_ref, w2_ref, uv_ref,
             wl2_ref, h2_ref, sc_ref, sk_ref):
        x1 = jnp.dot(g_ref[...], b1_ref[...],
                     preferred_element_type=jnp.float32)
        x1 = x1 + jnp.dot(x_ref[...], wl1_ref[...],
                          preferred_element_type=jnp.float32)
        x1 = x1 + bias_ref[...]
        x1 = jnp.where(x1 > 0, x1, jnp.exp(jnp.minimum(x1, 0.0)) - 1.0)
        h2_ref[...] = jnp.dot(x1, w2_ref[...],
                              preferred_element_type=jnp.float32)
        sc_ref[...] = jnp.dot(x1, uv_ref[...],
                              preferred_element_type=jnp.float32)
        sk_ref[...] = jnp.dot(x1, wl2_ref[...],
                              preferred_element_type=jnp.float32)

    bn = 1000
    return pl.pallas_call(
        body,
        grid=(N // bn,),
        in_specs=[pl.BlockSpec((bn, 256), lambda i: (i, 0)),
                  pl.BlockSpec((bn, 128), lambda i: (i, 0)),
                  pl.BlockSpec((256, 1024), lambda i: (0, 0)),
                  pl.BlockSpec((128, 1024), lambda i: (0, 0)),
                  pl.BlockSpec((1, 1024), lambda i: (0, 0)),
                  pl.BlockSpec((1024, 1024), lambda i: (0, 0)),
                  pl.BlockSpec((1024, 128), lambda i: (0, 0)),
                  pl.BlockSpec((1024, 1024), lambda i: (0, 0))],
        out_specs=[pl.BlockSpec((bn, 1024), lambda i: (i, 0)),
                   pl.BlockSpec((bn, 128), lambda i: (i, 0)),
                   pl.BlockSpec((bn, 1024), lambda i: (i, 0))],
        out_shape=[jax.ShapeDtypeStruct((N, 1024), jnp.float32),
                   jax.ShapeDtypeStruct((N, 128), jnp.float32),
                   jax.ShapeDtypeStruct((N, 1024), jnp.float32)],
    )(g1p, x128, B1p, Wl1p, bias1, W2, uv2p, Wl2)


def _kb(m2, sk2, bias2, W3p, uv3p, Wl3p):
    def body(m_ref, sk_ref, bias_ref, w3_ref, uv_ref, wl3_ref,
             h3_ref, sc_ref, sk3_ref):
        x2 = m_ref[...] + sk_ref[...] + bias_ref[...]
        x2 = jnp.where(x2 > 0, x2, jnp.exp(jnp.minimum(x2, 0.0)) - 1.0)
        h3_ref[...] = jnp.dot(x2, w3_ref[...],
                              preferred_element_type=jnp.float32)
        sc_ref[...] = jnp.dot(x2, uv_ref[...],
                              preferred_element_type=jnp.float32)
        sk3_ref[...] = jnp.dot(x2, wl3_ref[...],
                               preferred_element_type=jnp.float32)

    bn = 1000
    return pl.pallas_call(
        body,
        grid=(N // bn,),
        in_specs=[pl.BlockSpec((bn, 1024), lambda i: (i, 0)),
                  pl.BlockSpec((bn, 1024), lambda i: (i, 0)),
                  pl.BlockSpec((1, 1024), lambda i: (0, 0)),
                  pl.BlockSpec((1024, 768), lambda i: (0, 0)),
                  pl.BlockSpec((1024, 128), lambda i: (0, 0)),
                  pl.BlockSpec((1024, 128), lambda i: (0, 0))],
        out_specs=[pl.BlockSpec((bn, 768), lambda i: (i, 0)),
                   pl.BlockSpec((bn, 128), lambda i: (i, 0)),
                   pl.BlockSpec((bn, 128), lambda i: (i, 0))],
        out_shape=[jax.ShapeDtypeStruct((N, 768), jnp.float32),
                   jax.ShapeDtypeStruct((N, 128), jnp.float32),
                   jax.ShapeDtypeStruct((N, 128), jnp.float32)],
    )(m2, sk2, bias2, W3p, uv3p, Wl3p)


def _kc(m3, sk3, bias3):
    def body(m_ref, sk_ref, b_ref, o_ref):
        v = (m_ref[...] * jnp.float32(1.0 / H3) + sk_ref[...] + b_ref[...])
        o_ref[...] = v[:, :C_OUT]

    bn = 1000
    return pl.pallas_call(
        body,
        grid=(N // bn,),
        in_specs=[pl.BlockSpec((bn, 128), lambda i: (i, 0)),
                  pl.BlockSpec((bn, 128), lambda i: (i, 0)),
                  pl.BlockSpec((1, 128), lambda i: (0, 0))],
        out_specs=pl.BlockSpec((bn, C_OUT), lambda i: (i, 0)),
        out_shape=jax.ShapeDtypeStruct((N, C_OUT), jnp.float32),
    )(m3, sk3, bias3)


def kernel(x, edge_index, W1, a_src1, a_dst1, b1, Wl1, bl1, W2, a_src2,
           a_dst2, b2, Wl2, bl2, W3, a_src3, a_dst3, b3, Wl3, bl3):
    src = edge_index[0]
    dst = edge_index[1]

    perm = jnp.argsort(dst)
    srcs = src[perm]
    dsts = dst[perm]
    srcs_pad = jnp.zeros((EP,), jnp.int32).at[:E].set(srcs)
    dsts_pad = jnp.full((EP,), 2 * N_PAD, jnp.int32).at[:E].set(dsts)
    rp_pad = jnp.searchsorted(dsts, jnp.arange(RP_LEN),
                              side="left").astype(jnp.int32)

    def fold(W, a_s, a_d, heads, ch):
        Wr = W.reshape(W.shape[0], heads, ch)
        u = jnp.einsum("fkc,kc->fk", Wr, a_s)
        v = jnp.einsum("fkc,kc->fk", Wr, a_d)
        return jnp.concatenate([u, v], axis=1)

    uv1 = fold(W1, a_src1, a_dst1, H1, HID)
    uv2 = fold(W2, a_src2, a_dst2, H1, HID)
    uv3 = fold(W3, a_src3, a_dst3, H3, C_OUT)
    uv1p = jnp.zeros((128, 128), jnp.float32).at[:F_IN, :2 * H1].set(uv1)
    uv2p = jnp.zeros((1024, 128), jnp.float32).at[:, :2 * H1].set(uv2)
    uv3p = jnp.zeros((1024, 128), jnp.float32).at[:, :2 * H3].set(uv3)

    x128 = jnp.zeros((N, 128), jnp.float32).at[:, :F_IN].set(x)
    B1p = jnp.zeros((256, 1024), jnp.float32)
    W1r = W1.reshape(F_IN, H1, HID)
    for k in range(H1):
        B1p = B1p.at[k * 64:k * 64 + F_IN,
                     k * HID:(k + 1) * HID].set(W1r[:, k, :])
    Wl1p = jnp.zeros((128, 1024), jnp.float32).at[:F_IN, :].set(Wl1)
    bias1 = (b1 + bl1).reshape(1, 1024)
    bias2 = (b2 + bl2).reshape(1, 1024)
    W3p = jnp.zeros((1024, 768), jnp.float32).at[:, :H3 * C_OUT].set(W3)
    Wl3p = jnp.zeros((1024, 128), jnp.float32).at[:, :C_OUT].set(Wl3)
    bias3 = jnp.zeros((1, 128), jnp.float32).at[0, :C_OUT].set(b3 + bl3)

    sc1 = _k0(x128, uv1p)
    esc1 = _edge_scores(sc1[:, :H1], sc1[:, H1:2 * H1], srcs_pad, dsts_pad,
                        H1)
    g1 = _message(x128, esc1, rp_pad, srcs_pad, dsts_pad, H1, 128, 256, 1)
    g1p = g1[:N * 256].reshape(N, 256)

    h2, sc2, sk2 = _ka(g1p, x128, B1p, Wl1p, bias1, W2, uv2p, Wl2)
    esc2 = _edge_scores(sc2[:, :H1], sc2[:, H1:2 * H1], srcs_pad, dsts_pad,
                        H1)
    m2 = _message(h2, esc2, rp_pad, srcs_pad, dsts_pad, H1, 1024, 1024, 2)
    m2 = m2[:N * 1024].reshape(N, 1024)

    h3, sc3, sk3 = _kb(m2, sk2, bias2, W3p, uv3p, Wl3p)
    esc3a = _edge_scores(sc3[:, :3], sc3[:, H3:H3 + 3], srcs_pad, dsts_pad, 3)
    esc3b = _edge_scores(sc3[:, 3:H3], sc3[:, H3 + 3:2 * H3], srcs_pad,
                         dsts_pad, 3)
    esc3 = jnp.concatenate([esc3a, esc3b], axis=0)
    m3 = _message(h3, esc3, rp_pad, srcs_pad, dsts_pad, H3, 768, 128, 3)
    m3 = m3[:N * 128].reshape(N, 128)

    return _kc(m3, sk3, bias3)
